# baseline (device time: 18639 ns/iter reference)
import jax
import jax.numpy as jnp
from jax import lax
from jax.experimental import pallas as pl
from jax.experimental.pallas import tpu as pltpu

N_DEV = 4
NCHUNK = 4


def kernel(partial, resid, gamma):
    x = partial.reshape(partial.shape[-2], partial.shape[-1])
    m, n = x.shape
    half = m // 2
    quart = m // 4
    CHUNKS = [(0, 16), (16, 16), (32, 32), (64, 64)]
    gamma2d = gamma.reshape(1, n)

    def body(x_ref, resid_hbm, gamma_ref, out_ref,
             resid_v, rA1, rB1, rA2, rB2, send_sems, recv_sems, copy_sem):
        my = lax.axis_index("i")
        pa = my ^ 1
        pb = 3 - my

        kA1 = (my ^ (my >> 1)) & 1
        kB1 = my >> 1

        A_keep = kA1 * quart
        A_send = (1 - kA1) * quart
        B_keep = half + kB1 * quart
        B_send = half + (1 - kB1) * quart

        cp = pltpu.make_async_copy(resid_hbm, resid_v, copy_sem)
        cp.start()

        barrier_sem = pltpu.get_barrier_semaphore()
        for nbr in [pa, pb]:
            pl.semaphore_signal(
                barrier_sem, inc=1,
                device_id=(nbr,), device_id_type=pl.DeviceIdType.MESH,
            )
        pl.semaphore_wait(barrier_sem, 2)

        def rc(src_ref, src_start, rows, dst_ref, dst_start, peer, idx):
            return pltpu.make_async_remote_copy(
                src_ref=src_ref.at[pl.ds(src_start, rows), :],
                dst_ref=dst_ref.at[pl.ds(dst_start, rows), :],
                send_sem=send_sems.at[idx],
                recv_sem=recv_sems.at[idx],
                device_id=(peer,),
                device_id_type=pl.DeviceIdType.MESH,
            )

        s1a = [rc(x_ref, A_send + off, sz, rA1, off, pa, k)
               for k, (off, sz) in enumerate(CHUNKS)]
        s1b = [rc(x_ref, B_send + off, sz, rB1, off, pb, NCHUNK + k)
               for k, (off, sz) in enumerate(CHUNKS)]
        for k in range(NCHUNK):
            s1a[k].start()
            s1b[k].start()

        s2a = []
        s2b = []
        for k, (off, sz) in enumerate(CHUNKS):
            s1a[k].wait_recv()
            out_ref[pl.ds(A_keep + off, sz), :] = (
                x_ref[pl.ds(A_keep + off, sz), :] + rA1[pl.ds(off, sz), :]
            )
            r = rc(out_ref, A_keep + off, sz, rA2, off, pb, 2 * NCHUNK + k)
            r.start()
            s2a.append(r)

            s1b[k].wait_recv()
            out_ref[pl.ds(B_keep + off, sz), :] = (
                x_ref[pl.ds(B_keep + off, sz), :] + rB1[pl.ds(off, sz), :]
            )
            r = rc(out_ref, B_keep + off, sz, rB2, off, pa, 3 * NCHUNK + k)
            r.start()
            s2b.append(r)

        cp.wait()
        g = gamma_ref[0, :][None, :]

        g3a = []
        g3b = []
        for k, (off, sz) in enumerate(CHUNKS):
            s2a[k].wait_recv()
            start = A_keep + off
            y = (
                out_ref[pl.ds(start, sz), :]
                + rA2[pl.ds(off, sz), :]
                + resid_v[pl.ds(start, sz), :]
            )
            rms = jnp.sqrt(jnp.mean(y * y, axis=-1, keepdims=True) + 1e-6)
            out_ref[pl.ds(start, sz), :] = y / rms * g
            r = rc(out_ref, start, sz, out_ref, start, pa, 4 * NCHUNK + k)
            r.start()
            g3a.append(r)

            s2b[k].wait_recv()
            start = B_keep + off
            y = (
                out_ref[pl.ds(start, sz), :]
                + rB2[pl.ds(off, sz), :]
                + resid_v[pl.ds(start, sz), :]
            )
            rms = jnp.sqrt(jnp.mean(y * y, axis=-1, keepdims=True) + 1e-6)
            out_ref[pl.ds(start, sz), :] = y / rms * g
            r = rc(out_ref, start, sz, out_ref, start, pb, 5 * NCHUNK + k)
            r.start()
            g3b.append(r)

        for r in g3a + g3b:
            r.wait_recv()
        for r in s1a + s1b + s2a + s2b + g3a + g3b:
            r.wait_send()

    return pl.pallas_call(
        body,
        out_shape=jax.ShapeDtypeStruct((m, n), jnp.float32),
        in_specs=[
            pl.BlockSpec(memory_space=pltpu.VMEM),
            pl.BlockSpec(memory_space=pl.ANY),
            pl.BlockSpec(memory_space=pltpu.VMEM),
        ],
        out_specs=pl.BlockSpec(memory_space=pltpu.VMEM),
        scratch_shapes=[
            pltpu.VMEM((m, n), jnp.float32),
            pltpu.VMEM((quart, n), jnp.float32),
            pltpu.VMEM((quart, n), jnp.float32),
            pltpu.VMEM((quart, n), jnp.float32),
            pltpu.VMEM((quart, n), jnp.float32),
            pltpu.SemaphoreType.DMA((24,)),
            pltpu.SemaphoreType.DMA((24,)),
            pltpu.SemaphoreType.DMA,
        ],
        compiler_params=pltpu.CompilerParams(collective_id=0),
    )(x, resid, gamma2d)


# device time: 17940 ns/iter; 1.0390x vs baseline; 1.0390x over previous
import jax
import jax.numpy as jnp
from jax import lax
from jax.experimental import pallas as pl
from jax.experimental.pallas import tpu as pltpu

N_DEV = 4
NCHUNK = 4


def kernel(partial, resid, gamma):
    x = partial.reshape(partial.shape[-2], partial.shape[-1])
    m, n = x.shape
    half = m // 2
    quart = m // 4
    ch = quart // NCHUNK
    CHUNKS = [(k * ch, ch) for k in range(NCHUNK)]
    gamma2d = gamma.reshape(1, n)

    def body(x_ref, resid_hbm, gamma_ref, out_ref,
             resid_v, rA1, rB1, rA2, rB2, send_sems, recv_sems, copy_sem):
        my = lax.axis_index("i")
        pa = my ^ 1
        pb = 3 - my

        kA1 = (my ^ (my >> 1)) & 1
        kB1 = my >> 1

        A_keep = kA1 * quart
        A_send = (1 - kA1) * quart
        B_keep = half + kB1 * quart
        B_send = half + (1 - kB1) * quart

        cp = pltpu.make_async_copy(resid_hbm, resid_v, copy_sem)
        cp.start()

        barrier_sem = pltpu.get_barrier_semaphore()
        for nbr in [pa, pb]:
            pl.semaphore_signal(
                barrier_sem, inc=1,
                device_id=(nbr,), device_id_type=pl.DeviceIdType.MESH,
            )
        pl.semaphore_wait(barrier_sem, 2)

        def rc(src_ref, src_start, rows, dst_ref, dst_start, peer, idx):
            return pltpu.make_async_remote_copy(
                src_ref=src_ref.at[pl.ds(src_start, rows), :],
                dst_ref=dst_ref.at[pl.ds(dst_start, rows), :],
                send_sem=send_sems.at[idx],
                recv_sem=recv_sems.at[idx],
                device_id=(peer,),
                device_id_type=pl.DeviceIdType.MESH,
            )

        s1a = [rc(x_ref, A_send + off, sz, rA1, off, pa, k)
               for k, (off, sz) in enumerate(CHUNKS)]
        s1b = [rc(x_ref, B_send + off, sz, rB1, off, pb, NCHUNK + k)
               for k, (off, sz) in enumerate(CHUNKS)]
        for k in range(NCHUNK):
            s1a[k].start()
            s1b[k].start()

        s2a = []
        s2b = []
        for k, (off, sz) in enumerate(CHUNKS):
            s1a[k].wait_recv()
            out_ref[pl.ds(A_keep + off, sz), :] = (
                x_ref[pl.ds(A_keep + off, sz), :] + rA1[pl.ds(off, sz), :]
            )
            r = rc(out_ref, A_keep + off, sz, rA2, off, pb, 2 * NCHUNK + k)
            r.start()
            s2a.append(r)

            s1b[k].wait_recv()
            out_ref[pl.ds(B_keep + off, sz), :] = (
                x_ref[pl.ds(B_keep + off, sz), :] + rB1[pl.ds(off, sz), :]
            )
            r = rc(out_ref, B_keep + off, sz, rB2, off, pa, 3 * NCHUNK + k)
            r.start()
            s2b.append(r)

        cp.wait()
        g = gamma_ref[0, :][None, :]

        g3a = []
        g3b = []
        for k, (off, sz) in enumerate(CHUNKS):
            s2a[k].wait_recv()
            start = A_keep + off
            y = (
                out_ref[pl.ds(start, sz), :]
                + rA2[pl.ds(off, sz), :]
                + resid_v[pl.ds(start, sz), :]
            )
            rms = jnp.sqrt(jnp.mean(y * y, axis=-1, keepdims=True) + 1e-6)
            out_ref[pl.ds(start, sz), :] = y / rms * g
            r = rc(out_ref, start, sz, out_ref, start, pa, 4 * NCHUNK + k)
            r.start()
            g3a.append(r)

            s2b[k].wait_recv()
            start = B_keep + off
            y = (
                out_ref[pl.ds(start, sz), :]
                + rB2[pl.ds(off, sz), :]
                + resid_v[pl.ds(start, sz), :]
            )
            rms = jnp.sqrt(jnp.mean(y * y, axis=-1, keepdims=True) + 1e-6)
            out_ref[pl.ds(start, sz), :] = y / rms * g
            r = rc(out_ref, start, sz, out_ref, start, pb, 5 * NCHUNK + k)
            r.start()
            g3b.append(r)

        for r in g3a + g3b:
            r.wait_recv()
        for r in s1a + s1b + s2a + s2b + g3a + g3b:
            r.wait_send()

    return pl.pallas_call(
        body,
        out_shape=jax.ShapeDtypeStruct((m, n), jnp.float32),
        in_specs=[
            pl.BlockSpec(memory_space=pltpu.VMEM),
            pl.BlockSpec(memory_space=pl.ANY),
            pl.BlockSpec(memory_space=pltpu.VMEM),
        ],
        out_specs=pl.BlockSpec(memory_space=pltpu.VMEM),
        scratch_shapes=[
            pltpu.VMEM((m, n), jnp.float32),
            pltpu.VMEM((quart, n), jnp.float32),
            pltpu.VMEM((quart, n), jnp.float32),
            pltpu.VMEM((quart, n), jnp.float32),
            pltpu.VMEM((quart, n), jnp.float32),
            pltpu.SemaphoreType.DMA((24,)),
            pltpu.SemaphoreType.DMA((24,)),
            pltpu.SemaphoreType.DMA,
        ],
        compiler_params=pltpu.CompilerParams(collective_id=0),
    )(x, resid, gamma2d)
